# baseline (speedup 1.0000x reference)
import functools

import jax
import jax.numpy as jnp
from jax import lax
from jax.experimental import pallas as pl
from jax.experimental.pallas import tpu as pltpu
from jax.experimental.pallas import tpu_sc as plsc

B = 1024
L = 50
N_CODES = 100000
HIDDEN = 128

NC = 2
NS = 16
NW = NC * NS
BAGS_PER_W = B // NW
IDX_PER_W = BAGS_PER_W * L
CH_BAGS = 2
CH_ROWS = CH_BAGS * L
NCHUNK = BAGS_PER_W // CH_BAGS
LANES = 16
NGRP = HIDDEN // LANES


def _embbag_body(idx_hbm, table_hbm, out_hbm, idx_v, rows_v, acc_v, sem0, sem1):
    wid = lax.axis_index("s") * NC + lax.axis_index("c")
    pltpu.sync_copy(idx_hbm.at[wid], idx_v)

    sems = (sem0, sem1)

    def start(c):
        slot = c % 2
        return pltpu.async_copy(
            table_hbm.at[idx_v.at[c]],
            rows_v.at[slot], sems[slot])

    h = [None, None]
    h[0] = start(0)
    for c in range(NCHUNK):
        if c + 1 < NCHUNK:
            h[(c + 1) % 2] = start(c + 1)
        h[c % 2].wait()
        slot = c % 2
        for bag in range(CH_BAGS):
            def body(i, accs):
                out = list(accs)
                for u in range(5):
                    row = bag * L + i * 5 + u
                    for g in range(NGRP):
                        out[g] = out[g] + rows_v[slot, row,
                                                 pl.ds(g * LANES, LANES)]
                return tuple(out)
            accs = lax.fori_loop(
                0, L // 5, body,
                tuple(jnp.zeros((LANES,), jnp.float32) for _ in range(NGRP)))
            for g in range(NGRP):
                acc_v[c * CH_BAGS + bag, pl.ds(g * LANES, LANES)] = accs[g]
    pltpu.sync_copy(acc_v, out_hbm.at[pl.ds(wid * BAGS_PER_W, BAGS_PER_W)])


@jax.jit
def _embbag(idx_flat, table):
    mesh = plsc.VectorSubcoreMesh(core_axis_name="c", subcore_axis_name="s")
    f = functools.partial(
        pl.kernel,
        mesh=mesh,
        out_type=jax.ShapeDtypeStruct((B, HIDDEN), jnp.float32),
        scratch_types=[
            pltpu.VMEM((NCHUNK, CH_ROWS), jnp.int32),
            pltpu.VMEM((2, CH_ROWS, HIDDEN), jnp.float32),
            pltpu.VMEM((BAGS_PER_W, HIDDEN), jnp.float32),
            pltpu.SemaphoreType.DMA,
            pltpu.SemaphoreType.DMA,
        ],
    )(_embbag_body)
    return f(idx_flat, table)


BM = 2048
_NT = (((1,), (1,)), ((), ()))
_TN = (((0,), (0,)), ((), ()))


def _mlp_body(codes_ref, W1_ref, b1_ref, W2_ref, b2_ref, W3t_ref, b3r_ref,
              out_ref, h_ref):
    @pl.when(pl.program_id(0) == 0)
    def _():
        h1 = jnp.maximum(
            jnp.dot(codes_ref[...], W1_ref[...],
                    preferred_element_type=jnp.float32) + b1_ref[...], 0.0)
        h2 = jnp.maximum(
            jnp.dot(h1, W2_ref[...],
                    preferred_element_type=jnp.float32) + b2_ref[...], 0.0)
        h_ref[...] = h2

    out_ref[...] = lax.dot_general(
        W3t_ref[...], h_ref[...], _NT,
        preferred_element_type=jnp.float32) + lax.dot_general(
        b3r_ref[...], jnp.ones((8, B), jnp.float32), _TN,
        preferred_element_type=jnp.float32)


@jax.jit
def _mlp(codes, W1, b1, W2, b2, W3t, b3):
    nblk = pl.cdiv(N_CODES, BM)
    b3r = jnp.broadcast_to(b3 * 0.125, (8, N_CODES))
    out_t = pl.pallas_call(
        _mlp_body,
        grid=(nblk,),
        in_specs=[
            pl.BlockSpec((B, HIDDEN), lambda j: (0, 0)),
            pl.BlockSpec((HIDDEN, HIDDEN), lambda j: (0, 0)),
            pl.BlockSpec((1, HIDDEN), lambda j: (0, 0)),
            pl.BlockSpec((HIDDEN, HIDDEN), lambda j: (0, 0)),
            pl.BlockSpec((1, HIDDEN), lambda j: (0, 0)),
            pl.BlockSpec((BM, HIDDEN), lambda j: (j, 0)),
            pl.BlockSpec((8, BM), lambda j: (0, j)),
        ],
        out_specs=pl.BlockSpec((BM, B), lambda j: (j, 0)),
        out_shape=jax.ShapeDtypeStruct((N_CODES, B), jnp.float32),
        scratch_shapes=[pltpu.VMEM((B, HIDDEN), jnp.float32)],
    )(codes, W1, b1.reshape(1, HIDDEN), W2, b2.reshape(1, HIDDEN),
      W3t, b3r)
    return out_t.T


def kernel(batch_in, table, W1, b1, W2, b2, W3, b3):
    idx_flat = batch_in.astype(jnp.int32).reshape(NW, NCHUNK, CH_ROWS)
    codes = _embbag(idx_flat, table)
    return _mlp(codes, W1, b1, W2, b2, W3.T, b3)


# speedup vs baseline: 1.0390x; 1.0390x over previous
import functools

import jax
import jax.numpy as jnp
from jax import lax
from jax.experimental import pallas as pl
from jax.experimental.pallas import tpu as pltpu
from jax.experimental.pallas import tpu_sc as plsc

B = 1024
L = 50
N_CODES = 100000
HIDDEN = 128

NC = 2
NS = 16
NW = NC * NS
BAGS_PER_W = B // NW
IDX_PER_W = BAGS_PER_W * L
CH_BAGS = 2
CH_ROWS = CH_BAGS * L
NCHUNK = BAGS_PER_W // CH_BAGS
LANES = 16
NGRP = HIDDEN // LANES


def _embbag_body(idx_hbm, table_hbm, out_hbm, idx_v, rows_v, acc_v, sem0, sem1):
    wid = lax.axis_index("s") * NC + lax.axis_index("c")
    pltpu.sync_copy(idx_hbm.at[wid], idx_v)

    sems = (sem0, sem1)

    def start(c):
        slot = c % 2
        return pltpu.async_copy(
            table_hbm.at[idx_v.at[c]],
            rows_v.at[slot], sems[slot])

    h = [None, None]
    h[0] = start(0)
    for c in range(NCHUNK):
        if c + 1 < NCHUNK:
            h[(c + 1) % 2] = start(c + 1)
        h[c % 2].wait()
        slot = c % 2
        for bag in range(CH_BAGS):
            def body(r, accs):
                row = bag * L + r
                return tuple(
                    accs[g] + rows_v[slot, row, pl.ds(g * LANES, LANES)]
                    for g in range(NGRP))
            accs = lax.fori_loop(
                0, L, body,
                tuple(jnp.zeros((LANES,), jnp.float32) for _ in range(NGRP)))
            for g in range(NGRP):
                acc_v[c * CH_BAGS + bag, pl.ds(g * LANES, LANES)] = accs[g]
    pltpu.sync_copy(acc_v, out_hbm.at[pl.ds(wid * BAGS_PER_W, BAGS_PER_W)])


@jax.jit
def _embbag(idx_flat, table):
    mesh = plsc.VectorSubcoreMesh(core_axis_name="c", subcore_axis_name="s")
    f = functools.partial(
        pl.kernel,
        mesh=mesh,
        out_type=jax.ShapeDtypeStruct((B, HIDDEN), jnp.float32),
        scratch_types=[
            pltpu.VMEM((NCHUNK, CH_ROWS), jnp.int32),
            pltpu.VMEM((2, CH_ROWS, HIDDEN), jnp.float32),
            pltpu.VMEM((BAGS_PER_W, HIDDEN), jnp.float32),
            pltpu.SemaphoreType.DMA,
            pltpu.SemaphoreType.DMA,
        ],
    )(_embbag_body)
    return f(idx_flat, table)


BM = 2048
_NT = (((1,), (1,)), ((), ()))
_TN = (((0,), (0,)), ((), ()))


def _mlp_body(codes_ref, W1_ref, b1_ref, W2_ref, b2_ref, W3t_ref, b3r_ref,
              out_ref, h_ref):
    @pl.when(pl.program_id(0) == 0)
    def _():
        h1 = jnp.maximum(
            jnp.dot(codes_ref[...], W1_ref[...],
                    preferred_element_type=jnp.float32) + b1_ref[...], 0.0)
        h2 = jnp.maximum(
            jnp.dot(h1, W2_ref[...],
                    preferred_element_type=jnp.float32) + b2_ref[...], 0.0)
        h_ref[...] = h2

    out_ref[...] = lax.dot_general(
        W3t_ref[...], h_ref[...], _NT,
        preferred_element_type=jnp.float32) + lax.dot_general(
        b3r_ref[...], jnp.ones((1, B), jnp.float32), _TN,
        preferred_element_type=jnp.float32)


@jax.jit
def _mlp(codes, W1, b1, W2, b2, W3t, b3):
    nblk = pl.cdiv(N_CODES, BM)
    out_t = pl.pallas_call(
        _mlp_body,
        grid=(nblk,),
        in_specs=[
            pl.BlockSpec((B, HIDDEN), lambda j: (0, 0)),
            pl.BlockSpec((HIDDEN, HIDDEN), lambda j: (0, 0)),
            pl.BlockSpec((1, HIDDEN), lambda j: (0, 0)),
            pl.BlockSpec((HIDDEN, HIDDEN), lambda j: (0, 0)),
            pl.BlockSpec((1, HIDDEN), lambda j: (0, 0)),
            pl.BlockSpec((BM, HIDDEN), lambda j: (j, 0)),
            pl.BlockSpec((1, BM), lambda j: (0, j)),
        ],
        out_specs=pl.BlockSpec((BM, B), lambda j: (j, 0)),
        out_shape=jax.ShapeDtypeStruct((N_CODES, B), jnp.float32),
        scratch_shapes=[pltpu.VMEM((B, HIDDEN), jnp.float32)],
    )(codes, W1, b1.reshape(1, HIDDEN), W2, b2.reshape(1, HIDDEN),
      W3t, b3.reshape(1, N_CODES))
    return out_t.T


def kernel(batch_in, table, W1, b1, W2, b2, W3, b3):
    idx_flat = batch_in.astype(jnp.int32).reshape(NW, NCHUNK, CH_ROWS)
    codes = _embbag(idx_flat, table)
    return _mlp(codes, W1, b1, W2, b2, W3.T, b3)


# speedup vs baseline: 1.0532x; 1.0137x over previous
import functools

import jax
import jax.numpy as jnp
from jax import lax
from jax.experimental import pallas as pl
from jax.experimental.pallas import tpu as pltpu
from jax.experimental.pallas import tpu_sc as plsc

B = 1024
L = 50
N_CODES = 100000
HIDDEN = 128

NC = 2
NS = 16
NW = NC * NS
BAGS_PER_W = B // NW
IDX_PER_W = BAGS_PER_W * L
CH_BAGS = 2
CH_ROWS = CH_BAGS * L
NCHUNK = BAGS_PER_W // CH_BAGS
LANES = 16
NGRP = HIDDEN // LANES


def _embbag_body(idx_hbm, table_hbm, out_hbm, idx_v, rows_v, acc_v, sem0, sem1):
    wid = lax.axis_index("s") * NC + lax.axis_index("c")
    pltpu.sync_copy(idx_hbm.at[wid], idx_v)

    sems = (sem0, sem1)

    def start(c):
        slot = c % 2
        return pltpu.async_copy(
            table_hbm.at[idx_v.at[c]],
            rows_v.at[slot], sems[slot])

    h = [None, None]
    h[0] = start(0)
    for c in range(NCHUNK):
        if c + 1 < NCHUNK:
            h[(c + 1) % 2] = start(c + 1)
        h[c % 2].wait()
        slot = c % 2
        for bag in range(CH_BAGS):
            def body(r, accs):
                row = bag * L + r
                return tuple(
                    accs[g] + rows_v[slot, row, pl.ds(g * LANES, LANES)]
                    for g in range(NGRP))
            accs = lax.fori_loop(
                0, L, body,
                tuple(jnp.zeros((LANES,), jnp.float32) for _ in range(NGRP)))
            for g in range(NGRP):
                acc_v[c * CH_BAGS + bag, pl.ds(g * LANES, LANES)] = accs[g]
    pltpu.sync_copy(acc_v, out_hbm.at[pl.ds(wid * BAGS_PER_W, BAGS_PER_W)])


@jax.jit
def _embbag(idx_flat, table):
    mesh = plsc.VectorSubcoreMesh(core_axis_name="c", subcore_axis_name="s")
    f = functools.partial(
        pl.kernel,
        mesh=mesh,
        out_type=jax.ShapeDtypeStruct((B, HIDDEN), jnp.float32),
        scratch_types=[
            pltpu.VMEM((NCHUNK, CH_ROWS), jnp.int32),
            pltpu.VMEM((2, CH_ROWS, HIDDEN), jnp.float32),
            pltpu.VMEM((BAGS_PER_W, HIDDEN), jnp.float32),
            pltpu.SemaphoreType.DMA,
            pltpu.SemaphoreType.DMA,
        ],
    )(_embbag_body)
    return f(idx_flat, table)


BM = 3072
_NT = (((1,), (1,)), ((), ()))
_TN = (((0,), (0,)), ((), ()))


def _mlp_body(codes_ref, W1_ref, b1_ref, W2_ref, b2_ref, W3t_ref, b3r_ref,
              out_ref, h_ref):
    @pl.when(pl.program_id(0) == 0)
    def _():
        h1 = jnp.maximum(
            jnp.dot(codes_ref[...], W1_ref[...],
                    preferred_element_type=jnp.float32) + b1_ref[...], 0.0)
        h2 = jnp.maximum(
            jnp.dot(h1, W2_ref[...],
                    preferred_element_type=jnp.float32) + b2_ref[...], 0.0)
        h_ref[...] = h2

    out_ref[...] = lax.dot_general(
        W3t_ref[...], h_ref[...], _NT,
        preferred_element_type=jnp.float32) + lax.dot_general(
        b3r_ref[...], jnp.ones((1, B), jnp.float32), _TN,
        preferred_element_type=jnp.float32)


@jax.jit
def _mlp(codes, W1, b1, W2, b2, W3t, b3):
    nblk = pl.cdiv(N_CODES, BM)
    out_t = pl.pallas_call(
        _mlp_body,
        grid=(nblk,),
        in_specs=[
            pl.BlockSpec((B, HIDDEN), lambda j: (0, 0)),
            pl.BlockSpec((HIDDEN, HIDDEN), lambda j: (0, 0)),
            pl.BlockSpec((1, HIDDEN), lambda j: (0, 0)),
            pl.BlockSpec((HIDDEN, HIDDEN), lambda j: (0, 0)),
            pl.BlockSpec((1, HIDDEN), lambda j: (0, 0)),
            pl.BlockSpec((BM, HIDDEN), lambda j: (j, 0)),
            pl.BlockSpec((1, BM), lambda j: (0, j)),
        ],
        out_specs=pl.BlockSpec((BM, B), lambda j: (j, 0)),
        out_shape=jax.ShapeDtypeStruct((N_CODES, B), jnp.float32),
        scratch_shapes=[pltpu.VMEM((B, HIDDEN), jnp.float32)],
    )(codes, W1, b1.reshape(1, HIDDEN), W2, b2.reshape(1, HIDDEN),
      W3t, b3.reshape(1, N_CODES))
    return out_t.T


def kernel(batch_in, table, W1, b1, W2, b2, W3, b3):
    idx_flat = batch_in.astype(jnp.int32).reshape(NW, NCHUNK, CH_ROWS)
    codes = _embbag(idx_flat, table)
    return _mlp(codes, W1, b1, W2, b2, W3.T, b3)


# speedup vs baseline: 1.0554x; 1.0021x over previous
import functools

import jax
import jax.numpy as jnp
from jax import lax
from jax.experimental import pallas as pl
from jax.experimental.pallas import tpu as pltpu
from jax.experimental.pallas import tpu_sc as plsc

B = 1024
L = 50
N_CODES = 100000
HIDDEN = 128

NC = 2
NS = 16
NW = NC * NS
BAGS_PER_W = B // NW
IDX_PER_W = BAGS_PER_W * L
CH_BAGS = 2
CH_ROWS = CH_BAGS * L
NCHUNK = BAGS_PER_W // CH_BAGS
LANES = 16
NGRP = HIDDEN // LANES


def _embbag_body(idx_hbm, table_hbm, out_hbm, idx_v, rows_v, acc_v, sem0, sem1):
    wid = lax.axis_index("s") * NC + lax.axis_index("c")
    pltpu.sync_copy(idx_hbm.at[wid], idx_v)

    sems = (sem0, sem1)

    def start(c):
        slot = c % 2
        return pltpu.async_copy(
            table_hbm.at[idx_v.at[c]],
            rows_v.at[slot], sems[slot])

    h = [None, None]
    h[0] = start(0)
    for c in range(NCHUNK):
        if c + 1 < NCHUNK:
            h[(c + 1) % 2] = start(c + 1)
        h[c % 2].wait()
        slot = c % 2
        for bag in range(CH_BAGS):
            def body(r, accs):
                row = bag * L + r
                return tuple(
                    accs[g] + rows_v[slot, row, pl.ds(g * LANES, LANES)]
                    for g in range(NGRP))
            accs = lax.fori_loop(
                0, L, body,
                tuple(jnp.zeros((LANES,), jnp.float32) for _ in range(NGRP)))
            for g in range(NGRP):
                acc_v[c * CH_BAGS + bag, pl.ds(g * LANES, LANES)] = accs[g]
    pltpu.sync_copy(acc_v, out_hbm.at[pl.ds(wid * BAGS_PER_W, BAGS_PER_W)])


@jax.jit
def _embbag(idx_flat, table):
    mesh = plsc.VectorSubcoreMesh(core_axis_name="c", subcore_axis_name="s")
    f = functools.partial(
        pl.kernel,
        mesh=mesh,
        out_type=jax.ShapeDtypeStruct((B, HIDDEN), jnp.float32),
        scratch_types=[
            pltpu.VMEM((NCHUNK, CH_ROWS), jnp.int32),
            pltpu.VMEM((2, CH_ROWS, HIDDEN), jnp.float32),
            pltpu.VMEM((BAGS_PER_W, HIDDEN), jnp.float32),
            pltpu.SemaphoreType.DMA,
            pltpu.SemaphoreType.DMA,
        ],
    )(_embbag_body)
    return f(idx_flat, table)


BM = 4096
_NT = (((1,), (1,)), ((), ()))
_TN = (((0,), (0,)), ((), ()))


def _mlp_body(codes_ref, W1_ref, b1_ref, W2_ref, b2_ref, W3t_ref, b3r_ref,
              out_ref, h_ref):
    @pl.when(pl.program_id(0) == 0)
    def _():
        h1 = jnp.maximum(
            jnp.dot(codes_ref[...], W1_ref[...],
                    preferred_element_type=jnp.float32) + b1_ref[...], 0.0)
        h2 = jnp.maximum(
            jnp.dot(h1, W2_ref[...],
                    preferred_element_type=jnp.float32) + b2_ref[...], 0.0)
        h_ref[...] = h2

    out_ref[...] = lax.dot_general(
        W3t_ref[...], h_ref[...], _NT,
        preferred_element_type=jnp.float32) + lax.dot_general(
        b3r_ref[...], jnp.ones((1, B), jnp.float32), _TN,
        preferred_element_type=jnp.float32)


@jax.jit
def _mlp(codes, W1, b1, W2, b2, W3t, b3):
    nblk = pl.cdiv(N_CODES, BM)
    out_t = pl.pallas_call(
        _mlp_body,
        grid=(nblk,),
        in_specs=[
            pl.BlockSpec((B, HIDDEN), lambda j: (0, 0)),
            pl.BlockSpec((HIDDEN, HIDDEN), lambda j: (0, 0)),
            pl.BlockSpec((1, HIDDEN), lambda j: (0, 0)),
            pl.BlockSpec((HIDDEN, HIDDEN), lambda j: (0, 0)),
            pl.BlockSpec((1, HIDDEN), lambda j: (0, 0)),
            pl.BlockSpec((BM, HIDDEN), lambda j: (j, 0)),
            pl.BlockSpec((1, BM), lambda j: (0, j)),
        ],
        out_specs=pl.BlockSpec((BM, B), lambda j: (j, 0)),
        out_shape=jax.ShapeDtypeStruct((N_CODES, B), jnp.float32),
        scratch_shapes=[pltpu.VMEM((B, HIDDEN), jnp.float32)],
    )(codes, W1, b1.reshape(1, HIDDEN), W2, b2.reshape(1, HIDDEN),
      W3t, b3.reshape(1, N_CODES))
    return out_t.T


def kernel(batch_in, table, W1, b1, W2, b2, W3, b3):
    idx_flat = batch_in.astype(jnp.int32).reshape(NW, NCHUNK, CH_ROWS)
    codes = _embbag(idx_flat, table)
    return _mlp(codes, W1, b1, W2, b2, W3.T, b3)


# speedup vs baseline: 1.0766x; 1.0201x over previous
import functools

import jax
import jax.numpy as jnp
from jax import lax
from jax.experimental import pallas as pl
from jax.experimental.pallas import tpu as pltpu
from jax.experimental.pallas import tpu_sc as plsc

B = 1024
L = 50
N_CODES = 100000
HIDDEN = 128

NC = 2
NS = 16
NW = NC * NS
BAGS_PER_W = B // NW
IDX_PER_W = BAGS_PER_W * L
CH_BAGS = 2
CH_ROWS = CH_BAGS * L
NCHUNK = BAGS_PER_W // CH_BAGS
LANES = 16
NGRP = HIDDEN // LANES


NBUF = 4


def _embbag_body(idx_hbm, table_hbm, out_hbm, idx_v, rows_v, acc_v, *sems):
    wid = lax.axis_index("s") * NC + lax.axis_index("c")
    pltpu.sync_copy(idx_hbm.at[wid], idx_v)

    def start(c):
        slot = c % NBUF
        return pltpu.async_copy(
            table_hbm.at[idx_v.at[c]],
            rows_v.at[slot], sems[slot])

    h = [None] * NBUF
    for c in range(NBUF - 1):
        h[c] = start(c)
    for c in range(NCHUNK):
        if c + NBUF - 1 < NCHUNK:
            h[(c + NBUF - 1) % NBUF] = start(c + NBUF - 1)
        h[c % NBUF].wait()
        slot = c % NBUF
        for bag in range(CH_BAGS):
            def body(r, accs):
                row = bag * L + r
                return tuple(
                    accs[g] + rows_v[slot, row, pl.ds(g * LANES, LANES)]
                    for g in range(NGRP))
            accs = lax.fori_loop(
                0, L, body,
                tuple(jnp.zeros((LANES,), jnp.float32) for _ in range(NGRP)))
            for g in range(NGRP):
                acc_v[c * CH_BAGS + bag, pl.ds(g * LANES, LANES)] = accs[g]
    pltpu.sync_copy(acc_v, out_hbm.at[pl.ds(wid * BAGS_PER_W, BAGS_PER_W)])


@jax.jit
def _embbag(idx_flat, table):
    mesh = plsc.VectorSubcoreMesh(core_axis_name="c", subcore_axis_name="s")
    f = functools.partial(
        pl.kernel,
        mesh=mesh,
        out_type=jax.ShapeDtypeStruct((B, HIDDEN), jnp.float32),
        scratch_types=[
            pltpu.VMEM((NCHUNK, CH_ROWS), jnp.int32),
            pltpu.VMEM((NBUF, CH_ROWS, HIDDEN), jnp.float32),
            pltpu.VMEM((BAGS_PER_W, HIDDEN), jnp.float32),
        ] + [pltpu.SemaphoreType.DMA] * NBUF,
    )(_embbag_body)
    return f(idx_flat, table)


BM = 4096
_NT = (((1,), (1,)), ((), ()))
_TN = (((0,), (0,)), ((), ()))


def _mlp_body(codes_ref, W1_ref, b1_ref, W2_ref, b2_ref, W3t_ref, b3r_ref,
              out_ref, h_ref):
    @pl.when(pl.program_id(0) == 0)
    def _():
        h1 = jnp.maximum(
            jnp.dot(codes_ref[...], W1_ref[...],
                    preferred_element_type=jnp.float32) + b1_ref[...], 0.0)
        h2 = jnp.maximum(
            jnp.dot(h1, W2_ref[...],
                    preferred_element_type=jnp.float32) + b2_ref[...], 0.0)
        h_ref[...] = h2

    out_ref[...] = lax.dot_general(
        W3t_ref[...], h_ref[...], _NT,
        preferred_element_type=jnp.float32) + lax.dot_general(
        b3r_ref[...], jnp.ones((1, B), jnp.float32), _TN,
        preferred_element_type=jnp.float32)


@jax.jit
def _mlp(codes, W1, b1, W2, b2, W3t, b3):
    nblk = pl.cdiv(N_CODES, BM)
    out_t = pl.pallas_call(
        _mlp_body,
        grid=(nblk,),
        in_specs=[
            pl.BlockSpec((B, HIDDEN), lambda j: (0, 0)),
            pl.BlockSpec((HIDDEN, HIDDEN), lambda j: (0, 0)),
            pl.BlockSpec((1, HIDDEN), lambda j: (0, 0)),
            pl.BlockSpec((HIDDEN, HIDDEN), lambda j: (0, 0)),
            pl.BlockSpec((1, HIDDEN), lambda j: (0, 0)),
            pl.BlockSpec((BM, HIDDEN), lambda j: (j, 0)),
            pl.BlockSpec((1, BM), lambda j: (0, j)),
        ],
        out_specs=pl.BlockSpec((BM, B), lambda j: (j, 0)),
        out_shape=jax.ShapeDtypeStruct((N_CODES, B), jnp.float32),
        scratch_shapes=[pltpu.VMEM((B, HIDDEN), jnp.float32)],
    )(codes, W1, b1.reshape(1, HIDDEN), W2, b2.reshape(1, HIDDEN),
      W3t, b3.reshape(1, N_CODES))
    return out_t.T


def kernel(batch_in, table, W1, b1, W2, b2, W3, b3):
    idx_flat = batch_in.astype(jnp.int32).reshape(NW, NCHUNK, CH_ROWS)
    codes = _embbag(idx_flat, table)
    return _mlp(codes, W1, b1, W2, b2, W3.T, b3)


# speedup vs baseline: 1.0786x; 1.0018x over previous
import functools

import jax
import jax.numpy as jnp
from jax import lax
from jax.experimental import pallas as pl
from jax.experimental.pallas import tpu as pltpu
from jax.experimental.pallas import tpu_sc as plsc

B = 1024
L = 50
N_CODES = 100000
HIDDEN = 128

NC = 2
NS = 16
NW = NC * NS
BAGS_PER_W = B // NW
IDX_PER_W = BAGS_PER_W * L
CH_BAGS = 2
CH_ROWS = CH_BAGS * L
NCHUNK = BAGS_PER_W // CH_BAGS
LANES = 16
NGRP = HIDDEN // LANES


NBUF = 6


def _embbag_body(idx_hbm, table_hbm, out_hbm, idx_v, rows_v, acc_v, *sems):
    wid = lax.axis_index("s") * NC + lax.axis_index("c")
    pltpu.sync_copy(idx_hbm.at[wid], idx_v)

    def start(c):
        slot = c % NBUF
        return pltpu.async_copy(
            table_hbm.at[idx_v.at[c]],
            rows_v.at[slot], sems[slot])

    h = [None] * NBUF
    for c in range(NBUF - 1):
        h[c] = start(c)
    for c in range(NCHUNK):
        if c + NBUF - 1 < NCHUNK:
            h[(c + NBUF - 1) % NBUF] = start(c + NBUF - 1)
        h[c % NBUF].wait()
        slot = c % NBUF
        for bag in range(CH_BAGS):
            def body(r, accs):
                row = bag * L + r
                return tuple(
                    accs[g] + rows_v[slot, row, pl.ds(g * LANES, LANES)]
                    for g in range(NGRP))
            accs = lax.fori_loop(
                0, L, body,
                tuple(jnp.zeros((LANES,), jnp.float32) for _ in range(NGRP)))
            for g in range(NGRP):
                acc_v[c * CH_BAGS + bag, pl.ds(g * LANES, LANES)] = accs[g]
    pltpu.sync_copy(acc_v, out_hbm.at[pl.ds(wid * BAGS_PER_W, BAGS_PER_W)])


@jax.jit
def _embbag(idx_flat, table):
    mesh = plsc.VectorSubcoreMesh(core_axis_name="c", subcore_axis_name="s")
    f = functools.partial(
        pl.kernel,
        mesh=mesh,
        out_type=jax.ShapeDtypeStruct((B, HIDDEN), jnp.float32),
        scratch_types=[
            pltpu.VMEM((NCHUNK, CH_ROWS), jnp.int32),
            pltpu.VMEM((NBUF, CH_ROWS, HIDDEN), jnp.float32),
            pltpu.VMEM((BAGS_PER_W, HIDDEN), jnp.float32),
        ] + [pltpu.SemaphoreType.DMA] * NBUF,
    )(_embbag_body)
    return f(idx_flat, table)


BM = 4096
_NT = (((1,), (1,)), ((), ()))
_TN = (((0,), (0,)), ((), ()))


def _mlp_body(codes_ref, W1_ref, b1_ref, W2_ref, b2_ref, W3t_ref, b3r_ref,
              out_ref, h_ref):
    @pl.when(pl.program_id(0) == 0)
    def _():
        h1 = jnp.maximum(
            jnp.dot(codes_ref[...], W1_ref[...],
                    preferred_element_type=jnp.float32) + b1_ref[...], 0.0)
        h2 = jnp.maximum(
            jnp.dot(h1, W2_ref[...],
                    preferred_element_type=jnp.float32) + b2_ref[...], 0.0)
        h_ref[...] = h2

    out_ref[...] = lax.dot_general(
        W3t_ref[...], h_ref[...], _NT,
        preferred_element_type=jnp.float32) + lax.dot_general(
        b3r_ref[...], jnp.ones((1, B), jnp.float32), _TN,
        preferred_element_type=jnp.float32)


@jax.jit
def _mlp(codes, W1, b1, W2, b2, W3t, b3):
    nblk = pl.cdiv(N_CODES, BM)
    out_t = pl.pallas_call(
        _mlp_body,
        grid=(nblk,),
        in_specs=[
            pl.BlockSpec((B, HIDDEN), lambda j: (0, 0)),
            pl.BlockSpec((HIDDEN, HIDDEN), lambda j: (0, 0)),
            pl.BlockSpec((1, HIDDEN), lambda j: (0, 0)),
            pl.BlockSpec((HIDDEN, HIDDEN), lambda j: (0, 0)),
            pl.BlockSpec((1, HIDDEN), lambda j: (0, 0)),
            pl.BlockSpec((BM, HIDDEN), lambda j: (j, 0)),
            pl.BlockSpec((1, BM), lambda j: (0, j)),
        ],
        out_specs=pl.BlockSpec((BM, B), lambda j: (j, 0)),
        out_shape=jax.ShapeDtypeStruct((N_CODES, B), jnp.float32),
        scratch_shapes=[pltpu.VMEM((B, HIDDEN), jnp.float32)],
    )(codes, W1, b1.reshape(1, HIDDEN), W2, b2.reshape(1, HIDDEN),
      W3t, b3.reshape(1, N_CODES))
    return out_t.T


def kernel(batch_in, table, W1, b1, W2, b2, W3, b3):
    idx_flat = batch_in.astype(jnp.int32).reshape(NW, NCHUNK, CH_ROWS)
    codes = _embbag(idx_flat, table)
    return _mlp(codes, W1, b1, W2, b2, W3.T, b3)


# speedup vs baseline: 1.0796x; 1.0010x over previous
import functools

import jax
import jax.numpy as jnp
from jax import lax
from jax.experimental import pallas as pl
from jax.experimental.pallas import tpu as pltpu
from jax.experimental.pallas import tpu_sc as plsc

B = 1024
L = 50
N_CODES = 100000
HIDDEN = 128

NC = 2
NS = 16
NW = NC * NS
BAGS_PER_W = B // NW
IDX_PER_W = BAGS_PER_W * L
CH_BAGS = 2
CH_ROWS = CH_BAGS * L
NCHUNK = BAGS_PER_W // CH_BAGS
LANES = 16
NGRP = HIDDEN // LANES


NBUF = 6


def _embbag_body(idx_hbm, table_hbm, out_hbm, idx_v, rows_v, acc_v, *sems):
    wid = lax.axis_index("s") * NC + lax.axis_index("c")
    pltpu.sync_copy(idx_hbm.at[wid], idx_v)

    def start(c):
        slot = c % NBUF
        return pltpu.async_copy(
            table_hbm.at[idx_v.at[c]],
            rows_v.at[slot], sems[slot])

    h = [None] * NBUF
    for c in range(NBUF - 1):
        h[c] = start(c)
    for c in range(NCHUNK):
        if c + NBUF - 1 < NCHUNK:
            h[(c + NBUF - 1) % NBUF] = start(c + NBUF - 1)
        h[c % NBUF].wait()
        slot = c % NBUF
        for bag in range(CH_BAGS):
            def body(r, accs):
                row = bag * L + r
                return tuple(
                    accs[g] + rows_v[slot, row, pl.ds(g * LANES, LANES)]
                    for g in range(NGRP))
            accs = lax.fori_loop(
                0, L, body,
                tuple(jnp.zeros((LANES,), jnp.float32) for _ in range(NGRP)))
            for g in range(NGRP):
                acc_v[c * CH_BAGS + bag, pl.ds(g * LANES, LANES)] = accs[g]
    pltpu.sync_copy(acc_v, out_hbm.at[pl.ds(wid * BAGS_PER_W, BAGS_PER_W)])


@jax.jit
def _embbag(idx, table):
    mesh = plsc.VectorSubcoreMesh(core_axis_name="c", subcore_axis_name="s")
    f = functools.partial(
        pl.kernel,
        mesh=mesh,
        out_type=jax.ShapeDtypeStruct((B, HIDDEN), jnp.float32),
        scratch_types=[
            pltpu.VMEM((NCHUNK, CH_ROWS), jnp.int32),
            pltpu.VMEM((NBUF, CH_ROWS, HIDDEN), jnp.float32),
            pltpu.VMEM((BAGS_PER_W, HIDDEN), jnp.float32),
        ] + [pltpu.SemaphoreType.DMA] * NBUF,
    )(_embbag_body)
    return f(idx, table)


BM = 4096
_NT = (((1,), (1,)), ((), ()))
_TN = (((0,), (0,)), ((), ()))


def _mlp_body(codes_ref, W1_ref, b1_ref, W2_ref, b2_ref, W3t_ref, b3r_ref,
              out_ref, h_ref):
    @pl.when(pl.program_id(0) == 0)
    def _():
        h1 = jnp.maximum(
            jnp.dot(codes_ref[...], W1_ref[...],
                    preferred_element_type=jnp.float32) + b1_ref[...], 0.0)
        h2 = jnp.maximum(
            jnp.dot(h1, W2_ref[...],
                    preferred_element_type=jnp.float32) + b2_ref[...], 0.0)
        h_ref[...] = h2

    out_ref[...] = lax.dot_general(
        W3t_ref[...], h_ref[...], _NT,
        preferred_element_type=jnp.float32) + lax.dot_general(
        b3r_ref[...], jnp.ones((1, B), jnp.float32), _TN,
        preferred_element_type=jnp.float32)


@jax.jit
def _mlp(codes, W1, b1, W2, b2, W3t, b3):
    nblk = pl.cdiv(N_CODES, BM)
    out_t = pl.pallas_call(
        _mlp_body,
        grid=(nblk,),
        in_specs=[
            pl.BlockSpec((B, HIDDEN), lambda j: (0, 0)),
            pl.BlockSpec((HIDDEN, HIDDEN), lambda j: (0, 0)),
            pl.BlockSpec((1, HIDDEN), lambda j: (0, 0)),
            pl.BlockSpec((HIDDEN, HIDDEN), lambda j: (0, 0)),
            pl.BlockSpec((1, HIDDEN), lambda j: (0, 0)),
            pl.BlockSpec((BM, HIDDEN), lambda j: (j, 0)),
            pl.BlockSpec((1, BM), lambda j: (0, j)),
        ],
        out_specs=pl.BlockSpec((BM, B), lambda j: (j, 0)),
        out_shape=jax.ShapeDtypeStruct((N_CODES, B), jnp.float32),
        scratch_shapes=[pltpu.VMEM((B, HIDDEN), jnp.float32)],
    )(codes, W1, b1.reshape(1, HIDDEN), W2, b2.reshape(1, HIDDEN),
      W3t, b3.reshape(1, N_CODES))
    return out_t.T


def kernel(batch_in, table, W1, b1, W2, b2, W3, b3):
    idx = batch_in.astype(jnp.int32).reshape(NW, NCHUNK, CH_ROWS)
    codes = _embbag(idx, table)
    return _mlp(codes, W1, b1, W2, b2, W3.T, b3)


# speedup vs baseline: 1.0796x; 1.0000x over previous
import functools

import jax
import jax.numpy as jnp
from jax import lax
from jax.experimental import pallas as pl
from jax.experimental.pallas import tpu as pltpu
from jax.experimental.pallas import tpu_sc as plsc

B = 1024
L = 50
N_CODES = 100000
HIDDEN = 128

NC = 2
NS = 16
NW = NC * NS
BAGS_PER_W = B // NW
IDX_PER_W = BAGS_PER_W * L
CH_BAGS = 2
CH_ROWS = CH_BAGS * L
NCHUNK = BAGS_PER_W // CH_BAGS
LANES = 16
NGRP = HIDDEN // LANES


NBUF = 8


def _embbag_body(idx_hbm, table_hbm, out_hbm, idx_v, rows_v, acc_v, *sems):
    wid = lax.axis_index("s") * NC + lax.axis_index("c")
    pltpu.sync_copy(idx_hbm.at[wid], idx_v)

    def start(c):
        slot = c % NBUF
        return pltpu.async_copy(
            table_hbm.at[idx_v.at[c]],
            rows_v.at[slot], sems[slot])

    h = [None] * NBUF
    for c in range(NBUF - 1):
        h[c] = start(c)
    for c in range(NCHUNK):
        if c + NBUF - 1 < NCHUNK:
            h[(c + NBUF - 1) % NBUF] = start(c + NBUF - 1)
        h[c % NBUF].wait()
        slot = c % NBUF
        for bag in range(CH_BAGS):
            def body(r, accs):
                row = bag * L + r
                return tuple(
                    accs[g] + rows_v[slot, row, pl.ds(g * LANES, LANES)]
                    for g in range(NGRP))
            accs = lax.fori_loop(
                0, L, body,
                tuple(jnp.zeros((LANES,), jnp.float32) for _ in range(NGRP)))
            for g in range(NGRP):
                acc_v[c * CH_BAGS + bag, pl.ds(g * LANES, LANES)] = accs[g]
    pltpu.sync_copy(acc_v, out_hbm.at[pl.ds(wid * BAGS_PER_W, BAGS_PER_W)])


@jax.jit
def _embbag(idx, table):
    mesh = plsc.VectorSubcoreMesh(core_axis_name="c", subcore_axis_name="s")
    f = functools.partial(
        pl.kernel,
        mesh=mesh,
        out_type=jax.ShapeDtypeStruct((B, HIDDEN), jnp.float32),
        scratch_types=[
            pltpu.VMEM((NCHUNK, CH_ROWS), jnp.int32),
            pltpu.VMEM((NBUF, CH_ROWS, HIDDEN), jnp.float32),
            pltpu.VMEM((BAGS_PER_W, HIDDEN), jnp.float32),
        ] + [pltpu.SemaphoreType.DMA] * NBUF,
    )(_embbag_body)
    return f(idx, table)


BM = 4096
_NT = (((1,), (1,)), ((), ()))
_TN = (((0,), (0,)), ((), ()))


def _mlp_body(codes_ref, W1_ref, b1_ref, W2_ref, b2_ref, W3t_ref, b3r_ref,
              out_ref, h_ref):
    @pl.when(pl.program_id(0) == 0)
    def _():
        h1 = jnp.maximum(
            jnp.dot(codes_ref[...], W1_ref[...],
                    preferred_element_type=jnp.float32) + b1_ref[...], 0.0)
        h2 = jnp.maximum(
            jnp.dot(h1, W2_ref[...],
                    preferred_element_type=jnp.float32) + b2_ref[...], 0.0)
        h_ref[...] = h2

    out_ref[...] = lax.dot_general(
        W3t_ref[...], h_ref[...], _NT,
        preferred_element_type=jnp.float32) + lax.dot_general(
        b3r_ref[...], jnp.ones((1, B), jnp.float32), _TN,
        preferred_element_type=jnp.float32)


@jax.jit
def _mlp(codes, W1, b1, W2, b2, W3t, b3):
    nblk = pl.cdiv(N_CODES, BM)
    out_t = pl.pallas_call(
        _mlp_body,
        grid=(nblk,),
        in_specs=[
            pl.BlockSpec((B, HIDDEN), lambda j: (0, 0)),
            pl.BlockSpec((HIDDEN, HIDDEN), lambda j: (0, 0)),
            pl.BlockSpec((1, HIDDEN), lambda j: (0, 0)),
            pl.BlockSpec((HIDDEN, HIDDEN), lambda j: (0, 0)),
            pl.BlockSpec((1, HIDDEN), lambda j: (0, 0)),
            pl.BlockSpec((BM, HIDDEN), lambda j: (j, 0)),
            pl.BlockSpec((1, BM), lambda j: (0, j)),
        ],
        out_specs=pl.BlockSpec((BM, B), lambda j: (j, 0)),
        out_shape=jax.ShapeDtypeStruct((N_CODES, B), jnp.float32),
        scratch_shapes=[pltpu.VMEM((B, HIDDEN), jnp.float32)],
    )(codes, W1, b1.reshape(1, HIDDEN), W2, b2.reshape(1, HIDDEN),
      W3t, b3.reshape(1, N_CODES))
    return out_t.T


def kernel(batch_in, table, W1, b1, W2, b2, W3, b3):
    idx = batch_in.astype(jnp.int32).reshape(NW, NCHUNK, CH_ROWS)
    codes = _embbag(idx, table)
    return _mlp(codes, W1, b1, W2, b2, W3.T, b3)
